# baseline (speedup 1.0000x reference)
import functools

import jax
import jax.numpy as jnp
from jax import lax
from jax.experimental import pallas as pl
from jax.experimental.pallas import tpu as pltpu
from jax.experimental.pallas import tpu_sc as plsc

N = 10000
E = 320000
D = 128
H = 128
C_OUT = 10
G_SEG = 64

NC = 2
NS = 16
NW = NC * NS

NP = 10240
CHUNK = 128
NCHUNK = 80
NSTAGE = 40
EPW = NCHUNK * CHUNK
EP = NW * EPW
DUMP = 10008
RPT = NP // NS
NPT = NP // NW

BLK = 1024
GRID = NP // BLK


@functools.lru_cache(maxsize=None)
def _sc_kernels():
  mesh = plsc.VectorSubcoreMesh(core_axis_name="c", subcore_axis_name="s",
                                num_cores=NC, num_subcores=NS)

  @functools.partial(
      pl.kernel,
      out_type=jax.ShapeDtypeStruct((NC, NP, 16), jnp.float32),
      mesh=mesh,
      scratch_types=[
          pltpu.VMEM((NCHUNK, CHUNK), jnp.int32),
          pltpu.VMEM((CHUNK, 16), jnp.float32),
          pltpu.VMEM((CHUNK, 16), jnp.float32),
          pltpu.VMEM_SHARED((NP, 16), jnp.float32),
          pltpu.SemaphoreType.DMA,
      ],
  )
  def deg_kernel(dst_hbm, out_hbm, dst_v, ones_v, zb_v, deg_sh, sem):
    cid = lax.axis_index("c")
    sid = lax.axis_index("s")
    w = sid * NC + cid
    pltpu.sync_copy(dst_hbm.at[w], dst_v)
    one = jnp.full((16,), 1.0, jnp.float32)
    zero = jnp.zeros((16,), jnp.float32)
    for r in range(CHUNK):
      ones_v[r, pl.ds(0, 16)] = one
      zb_v[r, pl.ds(0, 16)] = zero
    for j in range(RPT // CHUNK):
      pltpu.sync_copy(zb_v, deg_sh.at[pl.ds(sid * RPT + j * CHUNK, CHUNK)])
    plsc.subcore_barrier()
    descs = [pltpu.async_copy(ones_v, deg_sh.at[dst_v.at[c]], sem, add=True)
             for c in range(NCHUNK)]
    for dsc in descs:
      dsc.wait()
    plsc.subcore_barrier()
    pltpu.sync_copy(deg_sh.at[pl.ds(sid * RPT, RPT)],
                    out_hbm.at[cid, pl.ds(sid * RPT, RPT)])

  @functools.partial(
      pl.kernel,
      out_type=jax.ShapeDtypeStruct((NC, NP, D), jnp.float32),
      mesh=mesh,
      scratch_types=[
          pltpu.VMEM((NSTAGE, CHUNK), jnp.int32),
          pltpu.VMEM((NCHUNK, CHUNK), jnp.int32),
          pltpu.VMEM((CHUNK, D), jnp.float32),
          pltpu.VMEM((CHUNK, D), jnp.float32),
          pltpu.VMEM_SHARED((NP, D), jnp.float32),
          pltpu.SemaphoreType.DMA,
          pltpu.SemaphoreType.DMA,
      ],
  )
  def agg_kernel(src_hbm, dst_hbm, xw_hbm, out_hbm,
                 src_v, dst_v, rows_a, rows_b, acc_sh, gsem, ssem):
    cid = lax.axis_index("c")
    sid = lax.axis_index("s")
    w = sid * NC + cid
    pltpu.sync_copy(src_hbm.at[w, pl.ds(0, NSTAGE)], src_v)
    pltpu.sync_copy(dst_hbm.at[w], dst_v)
    zero = jnp.zeros((16,), jnp.float32)
    for r in range(CHUNK):
      for j in range(D // 16):
        rows_a[r, pl.ds(j * 16, 16)] = zero
    for j in range(RPT // CHUNK):
      pltpu.sync_copy(rows_a, acc_sh.at[pl.ds(sid * RPT + j * CHUNK, CHUNK)])
    plsc.subcore_barrier()
    for c in range(NCHUNK):
      pltpu.async_copy(xw_hbm.at[src_v.at[c % NSTAGE]], rows_a, gsem).wait()
      if c == NSTAGE - 1:
        pltpu.sync_copy(src_hbm.at[w, pl.ds(NSTAGE, NSTAGE)], src_v)
      pltpu.sync_copy(rows_a, acc_sh.at[dst_v.at[c]], add=True)
    plsc.subcore_barrier()
    pltpu.sync_copy(acc_sh.at[pl.ds(sid * RPT, RPT)],
                    out_hbm.at[cid, pl.ds(sid * RPT, RPT)])

  @functools.partial(
      pl.kernel,
      out_type=jax.ShapeDtypeStruct((NC, G_SEG * D), jnp.float32),
      mesh=mesh,
      scratch_types=[
          pltpu.VMEM((NPT + 16,), jnp.int32),
          pltpu.VMEM((NPT * D,), jnp.float32),
          pltpu.VMEM(((G_SEG + 1) * D,), jnp.float32),
          pltpu.VMEM_SHARED((NS, G_SEG * D), jnp.float32),
          pltpu.VMEM((4 * D,), jnp.float32),
          pltpu.VMEM((4 * D,), jnp.float32),
      ],
  )
  def segmax_kernel(hflat_hbm, bat_hbm, out_hbm,
                    bat_v, h_v, loc_v, sh, acc_v, tmp_v):
    cid = lax.axis_index("c")
    sid = lax.axis_index("s")
    w = sid * NC + cid
    pltpu.sync_copy(bat_hbm.at[pl.ds(w * NPT, NPT)], bat_v.at[pl.ds(0, NPT)])
    pltpu.sync_copy(hflat_hbm.at[pl.ds(w * NPT * D, NPT * D)], h_v)
    ninf = jnp.full((16,), -jnp.inf, jnp.float32)
    for k in range((G_SEG + 1) * D // 16):
      loc_v[pl.ds(k * 16, 16)] = ninf

    def body(i, carry):
      b = bat_v[pl.ds(i, 16)][0]
      for j in range(D // 16):
        off = b * D + j * 16
        hoff = i * D + j * 16
        loc_v[pl.ds(off, 16)] = jnp.maximum(loc_v[pl.ds(off, 16)],
                                            h_v[pl.ds(hoff, 16)])
      return carry

    lax.fori_loop(0, NPT, body, 0)
    pltpu.sync_copy(loc_v.at[pl.ds(0, G_SEG * D)], sh.at[sid])
    plsc.subcore_barrier()
    seg_off = sid * 4 * D
    pltpu.sync_copy(sh.at[0, pl.ds(seg_off, 4 * D)], acc_v)
    for t in range(1, NS):
      pltpu.sync_copy(sh.at[t, pl.ds(seg_off, 4 * D)], tmp_v)
      for k in range(4 * D // 16):
        acc_v[pl.ds(k * 16, 16)] = jnp.maximum(acc_v[pl.ds(k * 16, 16)],
                                               tmp_v[pl.ds(k * 16, 16)])
    pltpu.sync_copy(acc_v, out_hbm.at[cid, pl.ds(seg_off, 4 * D)])

  return deg_kernel, agg_kernel, segmax_kernel


def _dinv_from(degp):
  deg = degp[0, :, 0] + degp[1, :, 0] + 1.0
  return lax.rsqrt(deg)[:, None]


def _elu(x):
  return jnp.where(x > 0, x, jnp.exp(x) - 1.0)


def _xw1_body(x_ref, w_ref, degp_ref, out_ref):
  dinv = _dinv_from(degp_ref[...])
  xw = jnp.dot(x_ref[...], w_ref[...], preferred_element_type=jnp.float32)
  out_ref[...] = dinv * xw


_xw1_call = pl.pallas_call(
    _xw1_body,
    grid=(GRID,),
    in_specs=[
        pl.BlockSpec((BLK, D), lambda i: (i, 0)),
        pl.BlockSpec((D, H), lambda i: (0, 0)),
        pl.BlockSpec((NC, BLK, 16), lambda i: (0, i, 0)),
    ],
    out_specs=pl.BlockSpec((BLK, H), lambda i: (i, 0)),
    out_shape=jax.ShapeDtypeStruct((NP, H), jnp.float32),
)


def _mid_body(accp_ref, xwp_ref, degp_ref, b_ref, w2_ref, out_ref):
  dinv = _dinv_from(degp_ref[...])
  a = accp_ref[...]
  pre = dinv * (a[0] + a[1] + xwp_ref[...]) + b_ref[...]
  h = _elu(pre)
  out_ref[...] = dinv * jnp.dot(h, w2_ref[...],
                                preferred_element_type=jnp.float32)


_mid_call = pl.pallas_call(
    _mid_body,
    grid=(GRID,),
    in_specs=[
        pl.BlockSpec((NC, BLK, H), lambda i: (0, i, 0)),
        pl.BlockSpec((BLK, H), lambda i: (i, 0)),
        pl.BlockSpec((NC, BLK, 16), lambda i: (0, i, 0)),
        pl.BlockSpec((1, H), lambda i: (0, 0)),
        pl.BlockSpec((H, H), lambda i: (0, 0)),
    ],
    out_specs=pl.BlockSpec((BLK, H), lambda i: (i, 0)),
    out_shape=jax.ShapeDtypeStruct((NP, H), jnp.float32),
)


def _h2_body(accp_ref, xwp_ref, degp_ref, b_ref, out_ref):
  dinv = _dinv_from(degp_ref[...])
  a = accp_ref[...]
  pre = dinv * (a[0] + a[1] + xwp_ref[...]) + b_ref[...]
  out_ref[...] = _elu(pre)


_h2_call = pl.pallas_call(
    _h2_body,
    grid=(GRID,),
    in_specs=[
        pl.BlockSpec((NC, BLK, H), lambda i: (0, i, 0)),
        pl.BlockSpec((BLK, H), lambda i: (i, 0)),
        pl.BlockSpec((NC, BLK, 16), lambda i: (0, i, 0)),
        pl.BlockSpec((1, H), lambda i: (0, 0)),
    ],
    out_specs=pl.BlockSpec((BLK, H), lambda i: (i, 0)),
    out_shape=jax.ShapeDtypeStruct((NP, H), jnp.float32),
)


def _head_body(pp_ref, ltw_ref, ltb_ref, out_ref):
  p = pp_ref[...]
  pooled = jnp.maximum(p[0], p[1])
  pooled = jnp.where(pooled < -3.0e38, 0.0, pooled)
  logits = jnp.dot(pooled, ltw_ref[...],
                   preferred_element_type=jnp.float32) + ltb_ref[...]
  m = jnp.max(logits, axis=1, keepdims=True)
  e = jnp.exp(logits - m)
  out_ref[...] = e / jnp.sum(e, axis=1, keepdims=True)


_head_call = pl.pallas_call(
    _head_body,
    out_shape=jax.ShapeDtypeStruct((G_SEG, C_OUT), jnp.float32),
)


def kernel(x, edge_index, batch, W1, b1, W2, b2, ltW, ltb):
  deg_kernel, agg_kernel, segmax_kernel = _sc_kernels()
  src = edge_index[0].astype(jnp.int32)
  dst = edge_index[1].astype(jnp.int32)
  pad_e = EP - E
  srcp = jnp.concatenate(
      [src, jnp.zeros((pad_e,), jnp.int32)]).reshape(NW, NCHUNK, CHUNK)
  dstp = jnp.concatenate(
      [dst, jnp.full((pad_e,), DUMP, jnp.int32)]).reshape(NW, NCHUNK, CHUNK)
  batp = jnp.concatenate(
      [batch.astype(jnp.int32), jnp.full((NP - N,), G_SEG, jnp.int32)])
  xp = jnp.pad(x, ((0, NP - N), (0, 0)))
  b1r = b1.reshape(1, H)
  b2r = b2.reshape(1, H)
  ltbr = ltb.reshape(1, C_OUT)

  degp = deg_kernel(dstp)
  xw1p = _xw1_call(xp, W1, degp)
  acc1 = agg_kernel(srcp, dstp, xw1p)
  xw2p = _mid_call(acc1, xw1p, degp, b1r, W2)
  acc2 = agg_kernel(srcp, dstp, xw2p)
  h2 = _h2_call(acc2, xw2p, degp, b2r)
  pooled = segmax_kernel(h2.reshape(-1), batp).reshape(NC, G_SEG, D)
  return _head_call(pooled, ltW, ltbr)


# speedup vs baseline: 1.1444x; 1.1444x over previous
import functools

import jax
import jax.numpy as jnp
from jax import lax
from jax.experimental import pallas as pl
from jax.experimental.pallas import tpu as pltpu
from jax.experimental.pallas import tpu_sc as plsc

N = 10000
E = 320000
D = 128
H = 128
C_OUT = 10
G_SEG = 64

NC = 2
NS = 16
NW = NC * NS

NP = 10240
CHUNK = 128
NCHUNK = 80
NSTAGE = 40
EPW = NCHUNK * CHUNK
EP = NW * EPW
DUMP = 10008
RPT = NP // NS
NPT = NP // NW

BLK = 1024
GRID = NP // BLK

TCH = NW * NCHUNK
N0 = 112
N1 = 48
MAXC = max(N0, N1)


@functools.lru_cache(maxsize=None)
def _sc_kernels():
  mesh = plsc.VectorSubcoreMesh(core_axis_name="c", subcore_axis_name="s",
                                num_cores=NC, num_subcores=NS)

  @functools.partial(
      pl.kernel,
      out_type=jax.ShapeDtypeStruct((NC, NP, 16), jnp.float32),
      mesh=mesh,
      scratch_types=[
          pltpu.VMEM((NCHUNK, CHUNK), jnp.int32),
          pltpu.VMEM((CHUNK, 16), jnp.float32),
          pltpu.VMEM((CHUNK, 16), jnp.float32),
          pltpu.VMEM_SHARED((NP, 16), jnp.float32),
          pltpu.SemaphoreType.DMA,
      ],
  )
  def deg_kernel(dst_hbm, out_hbm, dst_v, ones_v, zb_v, deg_sh, sem):
    cid = lax.axis_index("c")
    sid = lax.axis_index("s")
    w = sid * NC + cid
    pltpu.sync_copy(dst_hbm.at[w], dst_v)
    one = jnp.full((16,), 1.0, jnp.float32)
    zero = jnp.zeros((16,), jnp.float32)
    for r in range(CHUNK):
      ones_v[r, pl.ds(0, 16)] = one
      zb_v[r, pl.ds(0, 16)] = zero
    for j in range(RPT // CHUNK):
      pltpu.sync_copy(zb_v, deg_sh.at[pl.ds(sid * RPT + j * CHUNK, CHUNK)])
    plsc.subcore_barrier()
    descs = [pltpu.async_copy(ones_v, deg_sh.at[dst_v.at[c]], sem, add=True)
             for c in range(NCHUNK)]
    for dsc in descs:
      dsc.wait()
    plsc.subcore_barrier()
    pltpu.sync_copy(deg_sh.at[pl.ds(sid * RPT, RPT)],
                    out_hbm.at[cid, pl.ds(sid * RPT, RPT)])

  @functools.partial(
      pl.kernel,
      out_type=jax.ShapeDtypeStruct((NC, NP, D), jnp.float32),
      mesh=mesh,
      scratch_types=[
          pltpu.VMEM((MAXC, CHUNK), jnp.int32),
          pltpu.VMEM((MAXC, CHUNK), jnp.int32),
          pltpu.VMEM((CHUNK, D), jnp.float32),
          pltpu.VMEM_SHARED((NP, D), jnp.float32),
          pltpu.SemaphoreType.DMA,
      ],
  )
  def agg_kernel(src_hbm, dst_hbm, xw_hbm, out_hbm,
                 src_v, dst_v, rows_v, acc_sh, gsem):
    cid = lax.axis_index("c")
    sid = lax.axis_index("s")
    n_me = jnp.where(cid == 0, N0, N1)
    start = jnp.where(cid == 0, sid * N0, NS * N0 + sid * N1)
    pltpu.sync_copy(src_hbm.at[pl.ds(start, MAXC)], src_v)
    pltpu.sync_copy(dst_hbm.at[pl.ds(start, MAXC)], dst_v)
    zero = jnp.zeros((16,), jnp.float32)
    for r in range(CHUNK):
      for j in range(D // 16):
        rows_v[r, pl.ds(j * 16, 16)] = zero
    for j in range(RPT // CHUNK):
      pltpu.sync_copy(rows_v, acc_sh.at[pl.ds(sid * RPT + j * CHUNK, CHUNK)])
    plsc.subcore_barrier()
    def _chunk(c, carry):
      pltpu.async_copy(xw_hbm.at[src_v.at[c]], rows_v, gsem).wait()
      pltpu.sync_copy(rows_v, acc_sh.at[dst_v.at[c]], add=True)
      return carry

    lax.fori_loop(0, n_me, _chunk, 0)
    plsc.subcore_barrier()
    pltpu.sync_copy(acc_sh.at[pl.ds(sid * RPT, RPT)],
                    out_hbm.at[cid, pl.ds(sid * RPT, RPT)])

  @functools.partial(
      pl.kernel,
      out_type=jax.ShapeDtypeStruct((NC, G_SEG * D), jnp.float32),
      mesh=mesh,
      scratch_types=[
          pltpu.VMEM((NPT + 16,), jnp.int32),
          pltpu.VMEM((NPT * D,), jnp.float32),
          pltpu.VMEM(((G_SEG + 1) * D,), jnp.float32),
          pltpu.VMEM_SHARED((NS, G_SEG * D), jnp.float32),
          pltpu.VMEM((4 * D,), jnp.float32),
          pltpu.VMEM((4 * D,), jnp.float32),
      ],
  )
  def segmax_kernel(hflat_hbm, bat_hbm, out_hbm,
                    bat_v, h_v, loc_v, sh, acc_v, tmp_v):
    cid = lax.axis_index("c")
    sid = lax.axis_index("s")
    w = sid * NC + cid
    pltpu.sync_copy(bat_hbm.at[pl.ds(w * NPT, NPT)], bat_v.at[pl.ds(0, NPT)])
    pltpu.sync_copy(hflat_hbm.at[pl.ds(w * NPT * D, NPT * D)], h_v)
    ninf = jnp.full((16,), -jnp.inf, jnp.float32)
    for k in range((G_SEG + 1) * D // 16):
      loc_v[pl.ds(k * 16, 16)] = ninf

    def body(i, carry):
      b = bat_v[pl.ds(i, 16)][0]
      for j in range(D // 16):
        off = b * D + j * 16
        hoff = i * D + j * 16
        loc_v[pl.ds(off, 16)] = jnp.maximum(loc_v[pl.ds(off, 16)],
                                            h_v[pl.ds(hoff, 16)])
      return carry

    lax.fori_loop(0, NPT, body, 0)
    pltpu.sync_copy(loc_v.at[pl.ds(0, G_SEG * D)], sh.at[sid])
    plsc.subcore_barrier()
    seg_off = sid * 4 * D
    pltpu.sync_copy(sh.at[0, pl.ds(seg_off, 4 * D)], acc_v)
    for t in range(1, NS):
      pltpu.sync_copy(sh.at[t, pl.ds(seg_off, 4 * D)], tmp_v)
      for k in range(4 * D // 16):
        acc_v[pl.ds(k * 16, 16)] = jnp.maximum(acc_v[pl.ds(k * 16, 16)],
                                               tmp_v[pl.ds(k * 16, 16)])
    pltpu.sync_copy(acc_v, out_hbm.at[cid, pl.ds(seg_off, 4 * D)])

  return deg_kernel, agg_kernel, segmax_kernel


def _dinv_from(degp):
  deg = degp[0, :, 0] + degp[1, :, 0] + 1.0
  return lax.rsqrt(deg)[:, None]


def _elu(x):
  return jnp.where(x > 0, x, jnp.exp(x) - 1.0)


def _xw1_body(x_ref, w_ref, degp_ref, out_ref):
  dinv = _dinv_from(degp_ref[...])
  xw = jnp.dot(x_ref[...], w_ref[...], preferred_element_type=jnp.float32)
  out_ref[...] = dinv * xw


_xw1_call = pl.pallas_call(
    _xw1_body,
    grid=(GRID,),
    in_specs=[
        pl.BlockSpec((BLK, D), lambda i: (i, 0)),
        pl.BlockSpec((D, H), lambda i: (0, 0)),
        pl.BlockSpec((NC, BLK, 16), lambda i: (0, i, 0)),
    ],
    out_specs=pl.BlockSpec((BLK, H), lambda i: (i, 0)),
    out_shape=jax.ShapeDtypeStruct((NP, H), jnp.float32),
)


def _mid_body(accp_ref, xwp_ref, degp_ref, b_ref, w2_ref, out_ref):
  dinv = _dinv_from(degp_ref[...])
  a = accp_ref[...]
  pre = dinv * (a[0] + a[1] + xwp_ref[...]) + b_ref[...]
  h = _elu(pre)
  out_ref[...] = dinv * jnp.dot(h, w2_ref[...],
                                preferred_element_type=jnp.float32)


_mid_call = pl.pallas_call(
    _mid_body,
    grid=(GRID,),
    in_specs=[
        pl.BlockSpec((NC, BLK, H), lambda i: (0, i, 0)),
        pl.BlockSpec((BLK, H), lambda i: (i, 0)),
        pl.BlockSpec((NC, BLK, 16), lambda i: (0, i, 0)),
        pl.BlockSpec((1, H), lambda i: (0, 0)),
        pl.BlockSpec((H, H), lambda i: (0, 0)),
    ],
    out_specs=pl.BlockSpec((BLK, H), lambda i: (i, 0)),
    out_shape=jax.ShapeDtypeStruct((NP, H), jnp.float32),
)


def _h2_body(accp_ref, xwp_ref, degp_ref, b_ref, out_ref):
  dinv = _dinv_from(degp_ref[...])
  a = accp_ref[...]
  pre = dinv * (a[0] + a[1] + xwp_ref[...]) + b_ref[...]
  out_ref[...] = _elu(pre)


_h2_call = pl.pallas_call(
    _h2_body,
    grid=(GRID,),
    in_specs=[
        pl.BlockSpec((NC, BLK, H), lambda i: (0, i, 0)),
        pl.BlockSpec((BLK, H), lambda i: (i, 0)),
        pl.BlockSpec((NC, BLK, 16), lambda i: (0, i, 0)),
        pl.BlockSpec((1, H), lambda i: (0, 0)),
    ],
    out_specs=pl.BlockSpec((BLK, H), lambda i: (i, 0)),
    out_shape=jax.ShapeDtypeStruct((NP, H), jnp.float32),
)


def _head_body(pp_ref, ltw_ref, ltb_ref, out_ref):
  p = pp_ref[...]
  pooled = jnp.maximum(p[0], p[1])
  pooled = jnp.where(pooled < -3.0e38, 0.0, pooled)
  logits = jnp.dot(pooled, ltw_ref[...],
                   preferred_element_type=jnp.float32) + ltb_ref[...]
  m = jnp.max(logits, axis=1, keepdims=True)
  e = jnp.exp(logits - m)
  out_ref[...] = e / jnp.sum(e, axis=1, keepdims=True)


_head_call = pl.pallas_call(
    _head_body,
    out_shape=jax.ShapeDtypeStruct((G_SEG, C_OUT), jnp.float32),
)


def kernel(x, edge_index, batch, W1, b1, W2, b2, ltW, ltb):
  deg_kernel, agg_kernel, segmax_kernel = _sc_kernels()
  src = edge_index[0].astype(jnp.int32)
  dst = edge_index[1].astype(jnp.int32)
  pad_e = EP - E
  srcp = jnp.concatenate(
      [src, jnp.zeros((pad_e,), jnp.int32)]).reshape(NW, NCHUNK, CHUNK)
  dstp = jnp.concatenate(
      [dst, jnp.full((pad_e,), DUMP, jnp.int32)]).reshape(NW, NCHUNK, CHUNK)
  srcf = jnp.concatenate(
      [srcp.reshape(TCH, CHUNK), jnp.zeros((MAXC, CHUNK), jnp.int32)])
  dstf = jnp.concatenate(
      [dstp.reshape(TCH, CHUNK), jnp.full((MAXC, CHUNK), DUMP, jnp.int32)])
  batp = jnp.concatenate(
      [batch.astype(jnp.int32), jnp.full((NP - N,), G_SEG, jnp.int32)])
  xp = jnp.pad(x, ((0, NP - N), (0, 0)))
  b1r = b1.reshape(1, H)
  b2r = b2.reshape(1, H)
  ltbr = ltb.reshape(1, C_OUT)

  degp = deg_kernel(dstp)
  xw1p = _xw1_call(xp, W1, degp)
  acc1 = agg_kernel(srcf, dstf, xw1p)
  xw2p = _mid_call(acc1, xw1p, degp, b1r, W2)
  acc2 = agg_kernel(srcf, dstf, xw2p)
  h2 = _h2_call(acc2, xw2p, degp, b2r)
  pooled = segmax_kernel(h2.reshape(-1), batp).reshape(NC, G_SEG, D)
  return _head_call(pooled, ltW, ltbr)


# speedup vs baseline: 1.1845x; 1.0351x over previous
import functools

import jax
import jax.numpy as jnp
from jax import lax
from jax.experimental import pallas as pl
from jax.experimental.pallas import tpu as pltpu
from jax.experimental.pallas import tpu_sc as plsc

N = 10000
E = 320000
D = 128
H = 128
C_OUT = 10
G_SEG = 64

NC = 2
NS = 16
NW = NC * NS

NP = 10240
CHUNK = 128
NCHUNK = 80
NSTAGE = 40
EPW = NCHUNK * CHUNK
EP = NW * EPW
DUMP = 10008
RPT = NP // NS
NPT = NP // NW

BLK = 1024
GRID = NP // BLK

TCH = NW * NCHUNK
N0 = 120
N1 = 40
MAXC = max(N0, N1)


@functools.lru_cache(maxsize=None)
def _sc_kernels():
  mesh = plsc.VectorSubcoreMesh(core_axis_name="c", subcore_axis_name="s",
                                num_cores=NC, num_subcores=NS)

  @functools.partial(
      pl.kernel,
      out_type=jax.ShapeDtypeStruct((NC, NP, 16), jnp.float32),
      mesh=mesh,
      scratch_types=[
          pltpu.VMEM((NCHUNK, CHUNK), jnp.int32),
          pltpu.VMEM((CHUNK, 16), jnp.float32),
          pltpu.VMEM((CHUNK, 16), jnp.float32),
          pltpu.VMEM_SHARED((NP, 16), jnp.float32),
          pltpu.SemaphoreType.DMA,
      ],
  )
  def deg_kernel(dst_hbm, out_hbm, dst_v, ones_v, zb_v, deg_sh, sem):
    cid = lax.axis_index("c")
    sid = lax.axis_index("s")
    w = sid * NC + cid
    pltpu.sync_copy(dst_hbm.at[w], dst_v)
    one = jnp.full((16,), 1.0, jnp.float32)
    zero = jnp.zeros((16,), jnp.float32)
    for r in range(CHUNK):
      ones_v[r, pl.ds(0, 16)] = one
      zb_v[r, pl.ds(0, 16)] = zero
    for j in range(RPT // CHUNK):
      pltpu.sync_copy(zb_v, deg_sh.at[pl.ds(sid * RPT + j * CHUNK, CHUNK)])
    plsc.subcore_barrier()
    descs = [pltpu.async_copy(ones_v, deg_sh.at[dst_v.at[c]], sem, add=True)
             for c in range(NCHUNK)]
    for dsc in descs:
      dsc.wait()
    plsc.subcore_barrier()
    pltpu.sync_copy(deg_sh.at[pl.ds(sid * RPT, RPT)],
                    out_hbm.at[cid, pl.ds(sid * RPT, RPT)])

  @functools.partial(
      pl.kernel,
      out_type=jax.ShapeDtypeStruct((NC, NP, D), jnp.float32),
      mesh=mesh,
      scratch_types=[
          pltpu.VMEM((MAXC, CHUNK), jnp.int32),
          pltpu.VMEM((MAXC, CHUNK), jnp.int32),
          pltpu.VMEM((CHUNK, D), jnp.float32),
          pltpu.VMEM_SHARED((NP, D), jnp.float32),
          pltpu.SemaphoreType.DMA,
      ],
  )
  def agg_kernel(src_hbm, dst_hbm, xw_hbm, out_hbm,
                 src_v, dst_v, rows_v, acc_sh, gsem):
    cid = lax.axis_index("c")
    sid = lax.axis_index("s")
    n_me = jnp.where(cid == 0, N0, N1)
    start = jnp.where(cid == 0, sid * N0, NS * N0 + sid * N1)
    pltpu.sync_copy(src_hbm.at[pl.ds(start, MAXC)], src_v)
    pltpu.sync_copy(dst_hbm.at[pl.ds(start, MAXC)], dst_v)
    zero = jnp.zeros((16,), jnp.float32)
    for r in range(CHUNK):
      for j in range(D // 16):
        rows_v[r, pl.ds(j * 16, 16)] = zero
    for j in range(RPT // CHUNK):
      pltpu.sync_copy(rows_v, acc_sh.at[pl.ds(sid * RPT + j * CHUNK, CHUNK)])
    plsc.subcore_barrier()
    def _chunk(c, carry):
      pltpu.async_copy(xw_hbm.at[src_v.at[c]], rows_v, gsem).wait()
      pltpu.sync_copy(rows_v, acc_sh.at[dst_v.at[c]], add=True)
      return carry

    lax.fori_loop(0, n_me, _chunk, 0)
    plsc.subcore_barrier()
    pltpu.sync_copy(acc_sh.at[pl.ds(sid * RPT, RPT)],
                    out_hbm.at[cid, pl.ds(sid * RPT, RPT)])

  @functools.partial(
      pl.kernel,
      out_type=jax.ShapeDtypeStruct((NC, G_SEG * D), jnp.float32),
      mesh=mesh,
      scratch_types=[
          pltpu.VMEM((NPT + 16,), jnp.int32),
          pltpu.VMEM((NPT * D,), jnp.float32),
          pltpu.VMEM(((G_SEG + 1) * D,), jnp.float32),
          pltpu.VMEM_SHARED((NS, G_SEG * D), jnp.float32),
          pltpu.VMEM((4 * D,), jnp.float32),
          pltpu.VMEM((4 * D,), jnp.float32),
      ],
  )
  def segmax_kernel(hflat_hbm, bat_hbm, out_hbm,
                    bat_v, h_v, loc_v, sh, acc_v, tmp_v):
    cid = lax.axis_index("c")
    sid = lax.axis_index("s")
    w = sid * NC + cid
    pltpu.sync_copy(bat_hbm.at[pl.ds(w * NPT, NPT)], bat_v.at[pl.ds(0, NPT)])
    pltpu.sync_copy(hflat_hbm.at[pl.ds(w * NPT * D, NPT * D)], h_v)
    ninf = jnp.full((16,), -jnp.inf, jnp.float32)
    for k in range((G_SEG + 1) * D // 16):
      loc_v[pl.ds(k * 16, 16)] = ninf

    def body(i, carry):
      b = bat_v[pl.ds(i, 16)][0]
      for j in range(D // 16):
        off = b * D + j * 16
        hoff = i * D + j * 16
        loc_v[pl.ds(off, 16)] = jnp.maximum(loc_v[pl.ds(off, 16)],
                                            h_v[pl.ds(hoff, 16)])
      return carry

    lax.fori_loop(0, NPT, body, 0)
    pltpu.sync_copy(loc_v.at[pl.ds(0, G_SEG * D)], sh.at[sid])
    plsc.subcore_barrier()
    seg_off = sid * 4 * D
    pltpu.sync_copy(sh.at[0, pl.ds(seg_off, 4 * D)], acc_v)
    for t in range(1, NS):
      pltpu.sync_copy(sh.at[t, pl.ds(seg_off, 4 * D)], tmp_v)
      for k in range(4 * D // 16):
        acc_v[pl.ds(k * 16, 16)] = jnp.maximum(acc_v[pl.ds(k * 16, 16)],
                                               tmp_v[pl.ds(k * 16, 16)])
    pltpu.sync_copy(acc_v, out_hbm.at[cid, pl.ds(seg_off, 4 * D)])

  return deg_kernel, agg_kernel, segmax_kernel


def _dinv_from(degp):
  deg = degp[0, :, 0] + degp[1, :, 0] + 1.0
  return lax.rsqrt(deg)[:, None]


def _elu(x):
  return jnp.where(x > 0, x, jnp.exp(x) - 1.0)


def _xw1_body(x_ref, w_ref, degp_ref, out_ref):
  dinv = _dinv_from(degp_ref[...])
  xw = jnp.dot(x_ref[...], w_ref[...], preferred_element_type=jnp.float32)
  out_ref[...] = dinv * xw


_xw1_call = pl.pallas_call(
    _xw1_body,
    grid=(GRID,),
    in_specs=[
        pl.BlockSpec((BLK, D), lambda i: (i, 0)),
        pl.BlockSpec((D, H), lambda i: (0, 0)),
        pl.BlockSpec((NC, BLK, 16), lambda i: (0, i, 0)),
    ],
    out_specs=pl.BlockSpec((BLK, H), lambda i: (i, 0)),
    out_shape=jax.ShapeDtypeStruct((NP, H), jnp.float32),
)


def _mid_body(accp_ref, xwp_ref, degp_ref, b_ref, w2_ref, out_ref):
  dinv = _dinv_from(degp_ref[...])
  a = accp_ref[...]
  pre = dinv * (a[0] + a[1] + xwp_ref[...]) + b_ref[...]
  h = _elu(pre)
  out_ref[...] = dinv * jnp.dot(h, w2_ref[...],
                                preferred_element_type=jnp.float32)


_mid_call = pl.pallas_call(
    _mid_body,
    grid=(GRID,),
    in_specs=[
        pl.BlockSpec((NC, BLK, H), lambda i: (0, i, 0)),
        pl.BlockSpec((BLK, H), lambda i: (i, 0)),
        pl.BlockSpec((NC, BLK, 16), lambda i: (0, i, 0)),
        pl.BlockSpec((1, H), lambda i: (0, 0)),
        pl.BlockSpec((H, H), lambda i: (0, 0)),
    ],
    out_specs=pl.BlockSpec((BLK, H), lambda i: (i, 0)),
    out_shape=jax.ShapeDtypeStruct((NP, H), jnp.float32),
)


def _h2_body(accp_ref, xwp_ref, degp_ref, b_ref, out_ref):
  dinv = _dinv_from(degp_ref[...])
  a = accp_ref[...]
  pre = dinv * (a[0] + a[1] + xwp_ref[...]) + b_ref[...]
  out_ref[...] = _elu(pre)


_h2_call = pl.pallas_call(
    _h2_body,
    grid=(GRID,),
    in_specs=[
        pl.BlockSpec((NC, BLK, H), lambda i: (0, i, 0)),
        pl.BlockSpec((BLK, H), lambda i: (i, 0)),
        pl.BlockSpec((NC, BLK, 16), lambda i: (0, i, 0)),
        pl.BlockSpec((1, H), lambda i: (0, 0)),
    ],
    out_specs=pl.BlockSpec((BLK, H), lambda i: (i, 0)),
    out_shape=jax.ShapeDtypeStruct((NP, H), jnp.float32),
)


def _head_body(pp_ref, ltw_ref, ltb_ref, out_ref):
  p = pp_ref[...]
  pooled = jnp.maximum(p[0], p[1])
  pooled = jnp.where(pooled < -3.0e38, 0.0, pooled)
  logits = jnp.dot(pooled, ltw_ref[...],
                   preferred_element_type=jnp.float32) + ltb_ref[...]
  m = jnp.max(logits, axis=1, keepdims=True)
  e = jnp.exp(logits - m)
  out_ref[...] = e / jnp.sum(e, axis=1, keepdims=True)


_head_call = pl.pallas_call(
    _head_body,
    out_shape=jax.ShapeDtypeStruct((G_SEG, C_OUT), jnp.float32),
)


def kernel(x, edge_index, batch, W1, b1, W2, b2, ltW, ltb):
  deg_kernel, agg_kernel, segmax_kernel = _sc_kernels()
  src = edge_index[0].astype(jnp.int32)
  dst = edge_index[1].astype(jnp.int32)
  pad_e = EP - E
  srcp = jnp.concatenate(
      [src, jnp.zeros((pad_e,), jnp.int32)]).reshape(NW, NCHUNK, CHUNK)
  dstp = jnp.concatenate(
      [dst, jnp.full((pad_e,), DUMP, jnp.int32)]).reshape(NW, NCHUNK, CHUNK)
  srcf = jnp.concatenate(
      [srcp.reshape(TCH, CHUNK), jnp.zeros((MAXC, CHUNK), jnp.int32)])
  dstf = jnp.concatenate(
      [dstp.reshape(TCH, CHUNK), jnp.full((MAXC, CHUNK), DUMP, jnp.int32)])
  batp = jnp.concatenate(
      [batch.astype(jnp.int32), jnp.full((NP - N,), G_SEG, jnp.int32)])
  xp = jnp.pad(x, ((0, NP - N), (0, 0)))
  b1r = b1.reshape(1, H)
  b2r = b2.reshape(1, H)
  ltbr = ltb.reshape(1, C_OUT)

  degp = deg_kernel(dstp)
  xw1p = _xw1_call(xp, W1, degp)
  acc1 = agg_kernel(srcf, dstf, xw1p)
  xw2p = _mid_call(acc1, xw1p, degp, b1r, W2)
  acc2 = agg_kernel(srcf, dstf, xw2p)
  h2 = _h2_call(acc2, xw2p, degp, b2r)
  pooled = segmax_kernel(h2.reshape(-1), batp).reshape(NC, G_SEG, D)
  return _head_call(pooled, ltW, ltbr)


# speedup vs baseline: 1.2295x; 1.0380x over previous
import functools

import jax
import jax.numpy as jnp
from jax import lax
from jax.experimental import pallas as pl
from jax.experimental.pallas import tpu as pltpu
from jax.experimental.pallas import tpu_sc as plsc

N = 10000
E = 320000
D = 128
H = 128
C_OUT = 10
G_SEG = 64

NC = 2
NS = 16
NW = NC * NS

NP = 10240
CHUNK = 128
NCHUNK = 80
NSTAGE = 40
EPW = NCHUNK * CHUNK
EP = NW * EPW
DUMP = 10008
RPT = NP // NS
NPT = NP // NW

BLK = 1024
GRID = NP // BLK

TCH = NW * NCHUNK
N0 = 128
N1 = 32
MAXC = max(N0, N1)


@functools.lru_cache(maxsize=None)
def _sc_kernels():
  mesh = plsc.VectorSubcoreMesh(core_axis_name="c", subcore_axis_name="s",
                                num_cores=NC, num_subcores=NS)

  @functools.partial(
      pl.kernel,
      out_type=jax.ShapeDtypeStruct((NC, NP, 16), jnp.float32),
      mesh=mesh,
      scratch_types=[
          pltpu.VMEM((NCHUNK, CHUNK), jnp.int32),
          pltpu.VMEM((CHUNK, 16), jnp.float32),
          pltpu.VMEM((CHUNK, 16), jnp.float32),
          pltpu.VMEM_SHARED((NP, 16), jnp.float32),
          pltpu.SemaphoreType.DMA,
      ],
  )
  def deg_kernel(dst_hbm, out_hbm, dst_v, ones_v, zb_v, deg_sh, sem):
    cid = lax.axis_index("c")
    sid = lax.axis_index("s")
    w = sid * NC + cid
    pltpu.sync_copy(dst_hbm.at[w], dst_v)
    one = jnp.full((16,), 1.0, jnp.float32)
    zero = jnp.zeros((16,), jnp.float32)
    for r in range(CHUNK):
      ones_v[r, pl.ds(0, 16)] = one
      zb_v[r, pl.ds(0, 16)] = zero
    for j in range(RPT // CHUNK):
      pltpu.sync_copy(zb_v, deg_sh.at[pl.ds(sid * RPT + j * CHUNK, CHUNK)])
    plsc.subcore_barrier()
    descs = [pltpu.async_copy(ones_v, deg_sh.at[dst_v.at[c]], sem, add=True)
             for c in range(NCHUNK)]
    for dsc in descs:
      dsc.wait()
    plsc.subcore_barrier()
    pltpu.sync_copy(deg_sh.at[pl.ds(sid * RPT, RPT)],
                    out_hbm.at[cid, pl.ds(sid * RPT, RPT)])

  @functools.partial(
      pl.kernel,
      out_type=jax.ShapeDtypeStruct((NC, NP, D), jnp.float32),
      mesh=mesh,
      scratch_types=[
          pltpu.VMEM((MAXC, CHUNK), jnp.int32),
          pltpu.VMEM((MAXC, CHUNK), jnp.int32),
          pltpu.VMEM((CHUNK, D), jnp.float32),
          pltpu.VMEM_SHARED((NP, D), jnp.float32),
          pltpu.SemaphoreType.DMA,
      ],
  )
  def agg_kernel(src_hbm, dst_hbm, xw_hbm, out_hbm,
                 src_v, dst_v, rows_v, acc_sh, gsem):
    cid = lax.axis_index("c")
    sid = lax.axis_index("s")
    n_me = jnp.where(cid == 0, N0, N1)
    start = jnp.where(cid == 0, sid * N0, NS * N0 + sid * N1)
    pltpu.sync_copy(src_hbm.at[pl.ds(start, MAXC)], src_v)
    pltpu.sync_copy(dst_hbm.at[pl.ds(start, MAXC)], dst_v)
    zero = jnp.zeros((16,), jnp.float32)
    for r in range(CHUNK):
      for j in range(D // 16):
        rows_v[r, pl.ds(j * 16, 16)] = zero
    for j in range(RPT // CHUNK):
      pltpu.sync_copy(rows_v, acc_sh.at[pl.ds(sid * RPT + j * CHUNK, CHUNK)])
    plsc.subcore_barrier()
    def _chunk(c, carry):
      pltpu.async_copy(xw_hbm.at[src_v.at[c]], rows_v, gsem).wait()
      pltpu.sync_copy(rows_v, acc_sh.at[dst_v.at[c]], add=True)
      return carry

    lax.fori_loop(0, n_me, _chunk, 0)
    plsc.subcore_barrier()
    pltpu.sync_copy(acc_sh.at[pl.ds(sid * RPT, RPT)],
                    out_hbm.at[cid, pl.ds(sid * RPT, RPT)])

  @functools.partial(
      pl.kernel,
      out_type=jax.ShapeDtypeStruct((NC, G_SEG * D), jnp.float32),
      mesh=mesh,
      scratch_types=[
          pltpu.VMEM((NPT + 16,), jnp.int32),
          pltpu.VMEM((NPT * D,), jnp.float32),
          pltpu.VMEM(((G_SEG + 1) * D,), jnp.float32),
          pltpu.VMEM_SHARED((NS, G_SEG * D), jnp.float32),
          pltpu.VMEM((4 * D,), jnp.float32),
          pltpu.VMEM((4 * D,), jnp.float32),
      ],
  )
  def segmax_kernel(hflat_hbm, bat_hbm, out_hbm,
                    bat_v, h_v, loc_v, sh, acc_v, tmp_v):
    cid = lax.axis_index("c")
    sid = lax.axis_index("s")
    w = sid * NC + cid
    pltpu.sync_copy(bat_hbm.at[pl.ds(w * NPT, NPT)], bat_v.at[pl.ds(0, NPT)])
    pltpu.sync_copy(hflat_hbm.at[pl.ds(w * NPT * D, NPT * D)], h_v)
    ninf = jnp.full((16,), -jnp.inf, jnp.float32)
    for k in range((G_SEG + 1) * D // 16):
      loc_v[pl.ds(k * 16, 16)] = ninf

    def body(i, carry):
      b = bat_v[pl.ds(i, 16)][0]
      for j in range(D // 16):
        off = b * D + j * 16
        hoff = i * D + j * 16
        loc_v[pl.ds(off, 16)] = jnp.maximum(loc_v[pl.ds(off, 16)],
                                            h_v[pl.ds(hoff, 16)])
      return carry

    lax.fori_loop(0, NPT, body, 0)
    pltpu.sync_copy(loc_v.at[pl.ds(0, G_SEG * D)], sh.at[sid])
    plsc.subcore_barrier()
    seg_off = sid * 4 * D
    pltpu.sync_copy(sh.at[0, pl.ds(seg_off, 4 * D)], acc_v)
    for t in range(1, NS):
      pltpu.sync_copy(sh.at[t, pl.ds(seg_off, 4 * D)], tmp_v)
      for k in range(4 * D // 16):
        acc_v[pl.ds(k * 16, 16)] = jnp.maximum(acc_v[pl.ds(k * 16, 16)],
                                               tmp_v[pl.ds(k * 16, 16)])
    pltpu.sync_copy(acc_v, out_hbm.at[cid, pl.ds(seg_off, 4 * D)])

  return deg_kernel, agg_kernel, segmax_kernel


def _dinv_from(degp):
  deg = degp[0, :, 0] + degp[1, :, 0] + 1.0
  return lax.rsqrt(deg)[:, None]


def _elu(x):
  return jnp.where(x > 0, x, jnp.exp(x) - 1.0)


def _xw1_body(x_ref, w_ref, degp_ref, out_ref):
  dinv = _dinv_from(degp_ref[...])
  xw = jnp.dot(x_ref[...], w_ref[...], preferred_element_type=jnp.float32)
  out_ref[...] = dinv * xw


_xw1_call = pl.pallas_call(
    _xw1_body,
    grid=(GRID,),
    in_specs=[
        pl.BlockSpec((BLK, D), lambda i: (i, 0)),
        pl.BlockSpec((D, H), lambda i: (0, 0)),
        pl.BlockSpec((NC, BLK, 16), lambda i: (0, i, 0)),
    ],
    out_specs=pl.BlockSpec((BLK, H), lambda i: (i, 0)),
    out_shape=jax.ShapeDtypeStruct((NP, H), jnp.float32),
)


def _mid_body(accp_ref, xwp_ref, degp_ref, b_ref, w2_ref, out_ref):
  dinv = _dinv_from(degp_ref[...])
  a = accp_ref[...]
  pre = dinv * (a[0] + a[1] + xwp_ref[...]) + b_ref[...]
  h = _elu(pre)
  out_ref[...] = dinv * jnp.dot(h, w2_ref[...],
                                preferred_element_type=jnp.float32)


_mid_call = pl.pallas_call(
    _mid_body,
    grid=(GRID,),
    in_specs=[
        pl.BlockSpec((NC, BLK, H), lambda i: (0, i, 0)),
        pl.BlockSpec((BLK, H), lambda i: (i, 0)),
        pl.BlockSpec((NC, BLK, 16), lambda i: (0, i, 0)),
        pl.BlockSpec((1, H), lambda i: (0, 0)),
        pl.BlockSpec((H, H), lambda i: (0, 0)),
    ],
    out_specs=pl.BlockSpec((BLK, H), lambda i: (i, 0)),
    out_shape=jax.ShapeDtypeStruct((NP, H), jnp.float32),
)


def _h2_body(accp_ref, xwp_ref, degp_ref, b_ref, out_ref):
  dinv = _dinv_from(degp_ref[...])
  a = accp_ref[...]
  pre = dinv * (a[0] + a[1] + xwp_ref[...]) + b_ref[...]
  out_ref[...] = _elu(pre)


_h2_call = pl.pallas_call(
    _h2_body,
    grid=(GRID,),
    in_specs=[
        pl.BlockSpec((NC, BLK, H), lambda i: (0, i, 0)),
        pl.BlockSpec((BLK, H), lambda i: (i, 0)),
        pl.BlockSpec((NC, BLK, 16), lambda i: (0, i, 0)),
        pl.BlockSpec((1, H), lambda i: (0, 0)),
    ],
    out_specs=pl.BlockSpec((BLK, H), lambda i: (i, 0)),
    out_shape=jax.ShapeDtypeStruct((NP, H), jnp.float32),
)


def _head_body(pp_ref, ltw_ref, ltb_ref, out_ref):
  p = pp_ref[...]
  pooled = jnp.maximum(p[0], p[1])
  pooled = jnp.where(pooled < -3.0e38, 0.0, pooled)
  logits = jnp.dot(pooled, ltw_ref[...],
                   preferred_element_type=jnp.float32) + ltb_ref[...]
  m = jnp.max(logits, axis=1, keepdims=True)
  e = jnp.exp(logits - m)
  out_ref[...] = e / jnp.sum(e, axis=1, keepdims=True)


_head_call = pl.pallas_call(
    _head_body,
    out_shape=jax.ShapeDtypeStruct((G_SEG, C_OUT), jnp.float32),
)


def kernel(x, edge_index, batch, W1, b1, W2, b2, ltW, ltb):
  deg_kernel, agg_kernel, segmax_kernel = _sc_kernels()
  src = edge_index[0].astype(jnp.int32)
  dst = edge_index[1].astype(jnp.int32)
  pad_e = EP - E
  srcp = jnp.concatenate(
      [src, jnp.zeros((pad_e,), jnp.int32)]).reshape(NW, NCHUNK, CHUNK)
  dstp = jnp.concatenate(
      [dst, jnp.full((pad_e,), DUMP, jnp.int32)]).reshape(NW, NCHUNK, CHUNK)
  srcf = jnp.concatenate(
      [srcp.reshape(TCH, CHUNK), jnp.zeros((MAXC, CHUNK), jnp.int32)])
  dstf = jnp.concatenate(
      [dstp.reshape(TCH, CHUNK), jnp.full((MAXC, CHUNK), DUMP, jnp.int32)])
  batp = jnp.concatenate(
      [batch.astype(jnp.int32), jnp.full((NP - N,), G_SEG, jnp.int32)])
  xp = jnp.pad(x, ((0, NP - N), (0, 0)))
  b1r = b1.reshape(1, H)
  b2r = b2.reshape(1, H)
  ltbr = ltb.reshape(1, C_OUT)

  degp = deg_kernel(dstp)
  xw1p = _xw1_call(xp, W1, degp)
  acc1 = agg_kernel(srcf, dstf, xw1p)
  xw2p = _mid_call(acc1, xw1p, degp, b1r, W2)
  acc2 = agg_kernel(srcf, dstf, xw2p)
  h2 = _h2_call(acc2, xw2p, degp, b2r)
  pooled = segmax_kernel(h2.reshape(-1), batp).reshape(NC, G_SEG, D)
  return _head_call(pooled, ltW, ltbr)
